# baseline (device time: 40988 ns/iter reference)
import jax
import jax.numpy as jnp
from jax import lax
from jax.experimental import pallas as pl
from jax.experimental.pallas import tpu as pltpu


def kernel(x):
    m, n = x.shape

    def body(
        x_ref, out_ref, row_halo, col_halo, col_send, send_sems, recv_sems
    ):
        my_x = lax.axis_index("x")
        my_y = lax.axis_index("y")
        x_nbr = (1 - my_x, my_y)
        y_nbr = (my_x, 1 - my_y)

        barrier_sem = pltpu.get_barrier_semaphore()
        for nbr in (x_nbr, y_nbr):
            pl.semaphore_signal(
                barrier_sem, inc=1, device_id=nbr,
                device_id_type=pl.DeviceIdType.MESH,
            )
        pl.semaphore_wait(barrier_sem, 2)

        @pl.when(my_y == 0)
        def _():
            col_send[:, :] = x_ref[:, n - 1:n]

        @pl.when(my_y == 1)
        def _():
            col_send[:, :] = x_ref[:, 0:1]

        r_send = jnp.where(my_x == 0, m - 1, 0)

        rdma_row = pltpu.make_async_remote_copy(
            src_ref=x_ref.at[pl.ds(r_send, 1), :],
            dst_ref=row_halo,
            send_sem=send_sems.at[0],
            recv_sem=recv_sems.at[0],
            device_id=x_nbr,
            device_id_type=pl.DeviceIdType.MESH,
        )
        rdma_col = pltpu.make_async_remote_copy(
            src_ref=col_send,
            dst_ref=col_halo,
            send_sem=send_sems.at[1],
            recv_sem=recv_sems.at[1],
            device_id=y_nbr,
            device_id_type=pl.DeviceIdType.MESH,
        )
        rdma_row.start()
        rdma_col.start()

        out_ref[:, :] = x_ref[:, :]

        rdma_row.wait()
        rdma_col.wait()

        hrow = row_halo[:, :]
        hcol = col_halo[:, :]

        @pl.when(my_x == 1)
        def _():
            out_ref[0:1, :] = out_ref[0:1, :] + 0.125 * (hrow - x_ref[0:1, :])

        @pl.when(my_x == 0)
        def _():
            out_ref[m - 1:m, :] = out_ref[m - 1:m, :] + 0.125 * (hrow - x_ref[m - 1:m, :])

        @pl.when(my_y == 1)
        def _():
            out_ref[:, 0:1] = out_ref[:, 0:1] + 0.125 * (hcol - x_ref[:, n - 1:n])

        @pl.when(my_y == 0)
        def _():
            out_ref[:, n - 1:n] = out_ref[:, n - 1:n] + 0.125 * (hcol - x_ref[:, 0:1])

        @pl.when(my_x == 0)
        def _():
            out_ref[0:1, :] = x_ref[0:1, :]

        @pl.when(my_x == 1)
        def _():
            out_ref[m - 1:m, :] = x_ref[m - 1:m, :]

        @pl.when(my_y == 0)
        def _():
            out_ref[:, 0:1] = x_ref[:, 0:1]

        @pl.when(my_y == 1)
        def _():
            out_ref[:, n - 1:n] = x_ref[:, n - 1:n]

    return pl.pallas_call(
        body,
        out_shape=jax.ShapeDtypeStruct((m, n), x.dtype),
        in_specs=[pl.BlockSpec(memory_space=pltpu.VMEM)],
        out_specs=pl.BlockSpec(memory_space=pltpu.VMEM),
        scratch_shapes=[
            pltpu.VMEM((1, n), x.dtype),
            pltpu.VMEM((m, 1), x.dtype),
            pltpu.VMEM((m, 1), x.dtype),
            pltpu.SemaphoreType.DMA((2,)),
            pltpu.SemaphoreType.DMA((2,)),
        ],
        compiler_params=pltpu.CompilerParams(
            collective_id=0,
            vmem_limit_bytes=100 * 1024 * 1024,
        ),
    )(x)


# device time: 37294 ns/iter; 1.0991x vs baseline; 1.0991x over previous
import jax
import jax.numpy as jnp
from jax import lax
from jax.experimental import pallas as pl
from jax.experimental.pallas import tpu as pltpu

R = 256
S = 128


def kernel(x):
    m, n = x.shape
    K = m // R

    def body(
        x_hbm, out_hbm, xv, ov, estrip, row_halo, col_halo_t, col_send_t,
        send_sems, recv_sems, in_sems, strip_sem, out_sems, ostrip_sems,
    ):
        my_x = lax.axis_index("x")
        my_y = lax.axis_index("y")
        x_nbr = (1 - my_x, my_y)
        y_nbr = (my_x, 1 - my_y)

        r_send = jnp.where(my_x == 0, m - 1, 0)
        rdma_row = pltpu.make_async_remote_copy(
            src_ref=x_hbm.at[pl.ds(r_send, 1), :],
            dst_ref=row_halo,
            send_sem=send_sems.at[0],
            recv_sem=recv_sems.at[0],
            device_id=x_nbr,
            device_id_type=pl.DeviceIdType.MESH,
        )
        rdma_row.start()

        c_strip = jnp.where(my_y == 0, n - S, 0)
        strip_cp = pltpu.make_async_copy(
            x_hbm.at[:, pl.ds(c_strip, S)], estrip, strip_sem
        )
        strip_cp.start()

        in_copies = []
        for k in range(K):
            cp = pltpu.make_async_copy(
                x_hbm.at[pl.ds(k * R, R), :],
                xv.at[pl.ds(k * R, R), :],
                in_sems.at[k],
            )
            cp.start()
            in_copies.append(cp)

        barrier_sem = pltpu.get_barrier_semaphore()
        for nbr in (x_nbr, y_nbr):
            pl.semaphore_signal(
                barrier_sem, inc=1, device_id=nbr,
                device_id_type=pl.DeviceIdType.MESH,
            )

        strip_cp.wait()

        @pl.when(my_y == 0)
        def _():
            col_send_t[:, :] = jnp.transpose(estrip[:, S - 1:S], (1, 0))

        @pl.when(my_y == 1)
        def _():
            col_send_t[:, :] = jnp.transpose(estrip[:, 0:1], (1, 0))

        rdma_col = pltpu.make_async_remote_copy(
            src_ref=col_send_t,
            dst_ref=col_halo_t,
            send_sem=send_sems.at[1],
            recv_sem=recv_sems.at[1],
            device_id=y_nbr,
            device_id_type=pl.DeviceIdType.MESH,
        )
        rdma_col.start()

        def compute_chunk(k):
            i0 = k * R
            c = xv[i0:i0 + R, :]
            if k == 0:
                u = jnp.concatenate([xv[0:1, :], xv[0:R - 1, :]], axis=0)
            else:
                u = xv[i0 - 1:i0 + R - 1, :]
            if k == K - 1:
                d = jnp.concatenate([xv[i0 + 1:m, :], xv[m - 1:m, :]], axis=0)
            else:
                d = xv[i0 + 1:i0 + R + 1, :]
            l = pltpu.roll(c, 1, 1)
            r = pltpu.roll(c, n - 1, 1)
            ov[i0:i0 + R, :] = 0.5 * c + 0.125 * ((u + d) + (l + r))

        def out_chunk(k):
            cp = pltpu.make_async_copy(
                ov.at[pl.ds(k * R, R), :],
                out_hbm.at[pl.ds(k * R, R), :],
                out_sems.at[k],
            )
            cp.start()
            return cp

        out_copies = {}
        in_copies[0].wait()
        for k in range(1, K):
            in_copies[k].wait()
            compute_chunk(k - 1)
            if k - 1 not in (0, K - 1):
                out_copies[k - 1] = out_chunk(k - 1)
        compute_chunk(K - 1)

        rdma_row.wait()

        hrow = row_halo[:, :]

        @pl.when(my_x == 1)
        def _():
            ov[0:1, :] = ov[0:1, :] + 0.125 * (hrow - xv[0:1, :])

        @pl.when(my_x == 0)
        def _():
            ov[m - 1:m, :] = (
                ov[m - 1:m, :] + 0.125 * (hrow - xv[m - 1:m, :])
            )

        @pl.when(my_x == 0)
        def _():
            ov[0:1, :] = xv[0:1, :]

        @pl.when(my_x == 1)
        def _():
            ov[m - 1:m, :] = xv[m - 1:m, :]

        out_copies[0] = out_chunk(0)
        out_copies[K - 1] = out_chunk(K - 1)

        rdma_col.wait()
        hcol = jnp.transpose(col_halo_t[:, :], (1, 0))

        @pl.when(my_y == 1)
        def _():
            ov[:, 0:1] = ov[:, 0:1] + 0.125 * (hcol - xv[:, n - 1:n])

        @pl.when(my_y == 0)
        def _():
            ov[:, n - 1:n] = (
                ov[:, n - 1:n] + 0.125 * (hcol - xv[:, 0:1])
            )

        @pl.when(my_y == 0)
        def _():
            ov[:, 0:1] = xv[:, 0:1]

        @pl.when(my_y == 1)
        def _():
            ov[:, n - 1:n] = xv[:, n - 1:n]

        @pl.when(my_x == 0)
        def _():
            ov[0:1, 0:S] = xv[0:1, 0:S]
            ov[0:1, n - S:n] = xv[0:1, n - S:n]

        @pl.when(my_x == 1)
        def _():
            ov[m - 1:m, 0:S] = xv[m - 1:m, 0:S]
            ov[m - 1:m, n - S:n] = xv[m - 1:m, n - S:n]

        for k in range(K):
            out_copies[k].wait()
        lstrip_cp = pltpu.make_async_copy(
            ov.at[:, pl.ds(0, S)], out_hbm.at[:, pl.ds(0, S)], ostrip_sems.at[0]
        )
        rstrip_cp = pltpu.make_async_copy(
            ov.at[:, pl.ds(n - S, S)], out_hbm.at[:, pl.ds(n - S, S)],
            ostrip_sems.at[1],
        )
        lstrip_cp.start()
        rstrip_cp.start()
        lstrip_cp.wait()
        rstrip_cp.wait()

        pl.semaphore_wait(barrier_sem, 2)

    return pl.pallas_call(
        body,
        out_shape=jax.ShapeDtypeStruct((m, n), x.dtype),
        in_specs=[pl.BlockSpec(memory_space=pl.ANY)],
        out_specs=pl.BlockSpec(memory_space=pl.ANY),
        scratch_shapes=[
            pltpu.VMEM((m, n), x.dtype),
            pltpu.VMEM((m, n), x.dtype),
            pltpu.VMEM((m, S), x.dtype),
            pltpu.VMEM((1, n), x.dtype),
            pltpu.VMEM((1, m), x.dtype),
            pltpu.VMEM((1, m), x.dtype),
            pltpu.SemaphoreType.DMA((2,)),
            pltpu.SemaphoreType.DMA((2,)),
            pltpu.SemaphoreType.DMA((K,)),
            pltpu.SemaphoreType.DMA,
            pltpu.SemaphoreType.DMA((K,)),
            pltpu.SemaphoreType.DMA((2,)),
        ],
        compiler_params=pltpu.CompilerParams(
            collective_id=0,
            vmem_limit_bytes=100 * 1024 * 1024,
        ),
    )(x)
